# Initial kernel scaffold; baseline (speedup 1.0000x reference)
import jax
import jax.numpy as jnp
from jax.experimental import pallas as pl


def kernel(x, tables):
    raise NotImplementedError("write your pallas kernel here")


# speedup vs baseline: 1.0967x; 1.0967x over previous
import functools

import jax
import jax.numpy as jnp
from jax import lax
from jax.experimental import pallas as pl
from jax.experimental.pallas import tpu as pltpu
from jax.experimental.pallas import tpu_sc as plsc

B = 16384
F = 26
V = 100000
D = 32

NC = 2
NS = 16
NW = NC * NS

N_FLAT = B * F
ROWS = N_FLAT // 128
RPW = ROWS // NW


def _body(x_hbm, tbl_hbm, out_hbm, idx_v, buf_v, sem):
    c = lax.axis_index("c")
    s = lax.axis_index("s")
    wid = s * NC + c
    row0 = wid * RPW

    pltpu.sync_copy(x_hbm.at[pl.ds(row0, RPW)], idx_v)

    iota = lax.iota(jnp.int32, 16)

    def fix(j, carry):
        base = (row0 + j) * 128
        for g in range(8):
            sl = pl.ds(g * 16, 16)
            pos = base + g * 16 + iota
            f = lax.rem(pos, F)
            idx_v[j, sl] = idx_v[j, sl] + f * V
        return carry

    lax.fori_loop(0, RPW, fix, 0)

    def step(j, carry):
        pltpu.async_copy(tbl_hbm.at[idx_v.at[j]], buf_v, sem).wait()
        pltpu.sync_copy(buf_v, out_hbm.at[pl.ds((row0 + j) * 128, 128)])
        return carry

    lax.fori_loop(0, RPW, step, 0)


@jax.jit
def kernel(x, tables):
    x2d = x.astype(jnp.int32).reshape(ROWS, 128)
    tbl = tables.reshape(F * V, D)
    mesh = plsc.VectorSubcoreMesh(core_axis_name="c", subcore_axis_name="s",
                                  num_cores=NC, num_subcores=NS)
    out = pl.kernel(
        _body,
        out_type=jax.ShapeDtypeStruct((N_FLAT, D), jnp.float32),
        mesh=mesh,
        scratch_types=[
            pltpu.VMEM((RPW, 128), jnp.int32),
            pltpu.VMEM((128, D), jnp.float32),
            pltpu.SemaphoreType.DMA,
        ],
        compiler_params=pltpu.CompilerParams(use_tc_tiling_on_sc=False),
    )(x2d, tbl)
    return out.reshape(B, F, D)


# speedup vs baseline: 1.1507x; 1.0492x over previous
import jax
import jax.numpy as jnp
from jax import lax
from jax.experimental import pallas as pl
from jax.experimental.pallas import tpu as pltpu
from jax.experimental.pallas import tpu_sc as plsc

B = 16384
F = 26
V = 100000
D = 32

NC = 2
NS = 16
NW = NC * NS

N_FLAT = B * F
EPW = N_FLAT // NW
CHUNK = 512
NCH = EPW // CHUNK
HALF = NCH // 2


def _body(x_hbm, tbl_hbm, out_hbm, idx_v, buf_v, g0, g1, w0, w1):
    c = lax.axis_index("c")
    s = lax.axis_index("s")
    wid = s * NC + c
    e0 = wid * EPW

    pltpu.sync_copy(x_hbm.at[pl.ds(e0, EPW)], idx_v)

    iota = lax.iota(jnp.int32, 16)

    def fix(t):
        def fix_grp(g, carry):
            off = t * CHUNK + g * 16
            sl = pl.ds(off, 16)
            pos = e0 + off + iota
            f = lax.rem(pos, F)
            idx_v[sl] = idx_v[sl] + f * V
            return carry
        lax.fori_loop(0, CHUNK // 16, fix_grp, 0)

    def gstart(t, slot, sem):
        pltpu.async_copy(tbl_hbm.at[idx_v.at[pl.ds(t * CHUNK, CHUNK)]],
                         buf_v.at[slot], sem)

    def gwait(slot, sem):
        pltpu.make_async_copy(tbl_hbm.at[idx_v.at[pl.ds(0, CHUNK)]],
                              buf_v.at[slot], sem).wait()

    def wstart(t, slot, sem):
        pltpu.async_copy(buf_v.at[slot],
                         out_hbm.at[pl.ds(e0 + t * CHUNK, CHUNK)],
                         sem)

    def wwait(slot, sem):
        pltpu.make_async_copy(buf_v.at[slot],
                              out_hbm.at[pl.ds(e0, CHUNK)],
                              sem).wait()

    fix(0)
    gstart(0, 0, g0)
    fix(1)
    gstart(1, 1, g1)

    def step(i, carry):
        t0 = 2 * i
        gwait(0, g0)
        wstart(t0, 0, w0)
        gwait(1, g1)
        wstart(t0 + 1, 1, w1)

        @pl.when(i < HALF - 1)
        def _():
            fix(t0 + 2)
            wwait(0, w0)
            gstart(t0 + 2, 0, g0)
            fix(t0 + 3)
            wwait(1, w1)
            gstart(t0 + 3, 1, g1)

        return carry

    lax.fori_loop(0, HALF, step, 0)
    wwait(0, w0)
    wwait(1, w1)


@jax.jit
def kernel(x, tables):
    xf = x.astype(jnp.int32).reshape(N_FLAT)
    tbl = tables.reshape(F * V, D)
    mesh = plsc.VectorSubcoreMesh(core_axis_name="c", subcore_axis_name="s",
                                  num_cores=NC, num_subcores=NS)
    out = pl.kernel(
        _body,
        out_type=jax.ShapeDtypeStruct((N_FLAT, D), jnp.float32),
        mesh=mesh,
        scratch_types=[
            pltpu.VMEM((EPW,), jnp.int32),
            pltpu.VMEM((2, CHUNK, D), jnp.float32),
            pltpu.SemaphoreType.DMA,
            pltpu.SemaphoreType.DMA,
            pltpu.SemaphoreType.DMA,
            pltpu.SemaphoreType.DMA,
        ],
        compiler_params=pltpu.CompilerParams(use_tc_tiling_on_sc=False),
    )(xf, tbl)
    return out.reshape(B, F, D)


# speedup vs baseline: 1.4865x; 1.2918x over previous
import functools

import jax
import jax.numpy as jnp
from jax import lax
from jax.experimental import pallas as pl
from jax.experimental.pallas import tpu as pltpu
from jax.experimental.pallas import tpu_sc as plsc

B = 16384
F = 26
V = 100000
D = 32

NC = 2
NS = 16
NW = NC * NS

PR = F * V // 4
VCH = V
BPW = B // NW
CB = 256
NT = F * (BPW // CB)


def _pack_body(t_ref, o_ref):
    t = t_ref[...]
    for q in range(4):
        o_ref[:, q * D:(q + 1) * D] = t[:, q * (V // 4):(q + 1) * (V // 4)].T


def _pack(tbl_t):
    return pl.pallas_call(
        _pack_body,
        grid=(F,),
        in_specs=[pl.BlockSpec((D, VCH), lambda f: (f, 0))],
        out_specs=pl.BlockSpec((VCH // 4, 128), lambda f: (f, 0)),
        out_shape=jax.ShapeDtypeStruct((PR, 128), jnp.float32),
        compiler_params=pltpu.CompilerParams(
            vmem_limit_bytes=110 * 1024 * 1024),
    )(tbl_t)


def _gather_body(x_hbm, ptbl_hbm, out_hbm,
                 xk, idxb0, idxb1, laneb, gbuf, ebuf, g0, g1, w0, w1):
    c = lax.axis_index("c")
    s = lax.axis_index("s")
    wid = s * NC + c
    b0 = wid * BPW

    pltpu.sync_copy(x_hbm.at[pl.ds(b0 * F, BPW * F)], xk)

    iota = lax.iota(jnp.int32, 16)
    gsems = (g0, g1)
    wsems = (w0, w1)
    idxbs = (idxb0, idxb1)

    def build(t, slot):
        f = t // 2
        h = lax.rem(t, 2)

        def grp(g, carry):
            j = h * CB + g * 16 + iota
            v = plsc.load_gather(xk, [f + F * j])
            idxbs[slot][pl.ds(g * 16, 16)] = \
                f * (V // 4) + lax.rem(v, V // 4)
            laneb[slot, pl.ds(g * 16, 16)] = lax.div(v, V // 4) * D
            return carry

        lax.fori_loop(0, CB // 16, grp, 0)

    def gstart(slot):
        pltpu.async_copy(ptbl_hbm.at[idxbs[slot]], gbuf.at[slot],
                         gsems[slot])

    def gwait(slot):
        pltpu.make_async_copy(ptbl_hbm.at[idxbs[slot]], gbuf.at[slot],
                              gsems[slot]).wait()

    def extract(slot):
        def grp(g, carry):
            j = g * 16 + iota
            lj = laneb[slot, pl.ds(g * 16, 16)]
            for d in range(D):
                ebuf[slot, d, pl.ds(g * 16, 16)] = \
                    plsc.load_gather(gbuf.at[slot], [j, lj + d])
            return carry

        lax.fori_loop(0, CB // 16, grp, 0)

    def wstart(t, slot):
        f = t // 2
        h = lax.rem(t, 2)
        pltpu.async_copy(ebuf.at[slot],
                         out_hbm.at[f, :, pl.ds(b0 + h * CB, CB)],
                         wsems[slot])

    def wwait(slot):
        pltpu.make_async_copy(ebuf.at[slot],
                              out_hbm.at[0, :, pl.ds(b0, CB)],
                              wsems[slot]).wait()

    build(0, 0)
    gstart(0)
    build(1, 1)
    gstart(1)

    def step(i, carry):
        t0 = 2 * i
        gwait(0)
        extract(0)
        wstart(t0, 0)
        gwait(1)
        extract(1)
        wstart(t0 + 1, 1)

        @pl.when(i < NT // 2 - 1)
        def _():
            build(t0 + 2, 0)
            wwait(0)
            gstart(0)
            build(t0 + 3, 1)
            wwait(1)
            gstart(1)

        return carry

    lax.fori_loop(0, NT // 2, step, 0)
    wwait(0)
    wwait(1)


def _gather(xf, ptbl):
    mesh = plsc.VectorSubcoreMesh(core_axis_name="c", subcore_axis_name="s",
                                  num_cores=NC, num_subcores=NS)
    return pl.kernel(
        _gather_body,
        out_type=jax.ShapeDtypeStruct((F, D, B), jnp.float32),
        mesh=mesh,
        scratch_types=[
            pltpu.VMEM((BPW * F,), jnp.int32),
            pltpu.VMEM((CB,), jnp.int32),
            pltpu.VMEM((CB,), jnp.int32),
            pltpu.VMEM((2, CB), jnp.int32),
            pltpu.VMEM((2, CB, 128), jnp.float32),
            pltpu.VMEM((2, D, CB), jnp.float32),
            pltpu.SemaphoreType.DMA,
            pltpu.SemaphoreType.DMA,
            pltpu.SemaphoreType.DMA,
            pltpu.SemaphoreType.DMA,
        ],
        compiler_params=pltpu.CompilerParams(use_tc_tiling_on_sc=True,
                                             needs_layout_passes=False),
    )(xf, ptbl)


@jax.jit
def kernel(x, tables):
    xf = x.astype(jnp.int32).reshape(B * F)
    tbl_t = jnp.transpose(tables, (0, 2, 1)).reshape(F * D, V)
    ptbl = _pack(tbl_t)
    out_fdb = _gather(xf, ptbl)
    return jnp.transpose(out_fdb, (2, 0, 1))


# speedup vs baseline: 1.5575x; 1.0478x over previous
import functools

import jax
import jax.numpy as jnp
from jax import lax
from jax.experimental import pallas as pl
from jax.experimental.pallas import tpu as pltpu
from jax.experimental.pallas import tpu_sc as plsc

B = 16384
F = 26
V = 100000
D = 32

NC = 2
NS = 16
NW = NC * NS

PR = F * V // 4
VCH = V
BPW = B // NW
CB = 128
NT = F * (BPW // CB)
NSL = 4
NGRP = NT // NSL


def _pack_body(t_ref, o_ref):
    t = t_ref[...]
    for q in range(4):
        o_ref[:, q * D:(q + 1) * D] = t[:, q * (V // 4):(q + 1) * (V // 4)].T


def _pack(tbl_t):
    return pl.pallas_call(
        _pack_body,
        grid=(F,),
        in_specs=[pl.BlockSpec((D, VCH), lambda f: (f, 0))],
        out_specs=pl.BlockSpec((VCH // 4, 128), lambda f: (f, 0)),
        out_shape=jax.ShapeDtypeStruct((PR, 128), jnp.float32),
        compiler_params=pltpu.CompilerParams(
            vmem_limit_bytes=110 * 1024 * 1024),
    )(tbl_t)


def _gather_body(x_hbm, ptbl_hbm, out_hbm,
                 xk, idxb0, idxb1, idxb2, idxb3, laneb, gbuf, ebuf,
                 g0, g1, g2, g3, w0, w1, w2, w3):
    c = lax.axis_index("c")
    s = lax.axis_index("s")
    wid = s * NC + c
    b0 = wid * BPW

    pltpu.sync_copy(x_hbm.at[pl.ds(b0 * F, BPW * F)], xk)

    iota = lax.iota(jnp.int32, 16)
    gsems = (g0, g1, g2, g3)
    wsems = (w0, w1, w2, w3)
    idxbs = (idxb0, idxb1, idxb2, idxb3)

    def build(t, slot):
        f = t // NSL
        h = lax.rem(t, NSL)

        def grp(g, carry):
            j = h * CB + g * 16 + iota
            v = plsc.load_gather(xk, [f + F * j])
            idxbs[slot][pl.ds(g * 16, 16)] = \
                f * (V // 4) + lax.rem(v, V // 4)
            laneb[slot, pl.ds(g * 16, 16)] = lax.div(v, V // 4) * D
            return carry

        lax.fori_loop(0, CB // 16, grp, 0)

    def gstart(slot):
        pltpu.async_copy(ptbl_hbm.at[idxbs[slot]], gbuf.at[slot],
                         gsems[slot])

    def gwait(slot):
        pltpu.make_async_copy(ptbl_hbm.at[idxbs[slot]], gbuf.at[slot],
                              gsems[slot]).wait()

    def extract(slot):
        def grp(g, carry):
            j = g * 16 + iota
            lj = laneb[slot, pl.ds(g * 16, 16)]
            for d in range(D):
                ebuf[slot, d, pl.ds(g * 16, 16)] = \
                    plsc.load_gather(gbuf.at[slot], [j, lj + d])
            return carry

        lax.fori_loop(0, CB // 16, grp, 0)

    def wstart(t, slot):
        f = t // NSL
        h = lax.rem(t, NSL)
        pltpu.async_copy(ebuf.at[slot],
                         out_hbm.at[f, :, pl.ds(b0 + h * CB, CB)],
                         wsems[slot])

    def wwait(slot):
        pltpu.make_async_copy(ebuf.at[slot],
                              out_hbm.at[0, :, pl.ds(b0, CB)],
                              wsems[slot]).wait()

    for sl in range(NSL):
        build(sl, sl)
        gstart(sl)

    def group(gi, carry):
        for sl in range(NSL):
            t = NSL * gi + sl
            gwait(sl)

            @pl.when(gi >= 1)
            def _():
                wwait(sl)

            extract(sl)
            wstart(t, sl)

            @pl.when(gi < NGRP - 1)
            def _():
                build(t + NSL, sl)
                gstart(sl)

        return carry

    lax.fori_loop(0, NGRP, group, 0)
    for sl in range(NSL):
        wwait(sl)


def _gather(xf, ptbl):
    mesh = plsc.VectorSubcoreMesh(core_axis_name="c", subcore_axis_name="s",
                                  num_cores=NC, num_subcores=NS)
    return pl.kernel(
        _gather_body,
        out_type=jax.ShapeDtypeStruct((F, D, B), jnp.float32),
        mesh=mesh,
        scratch_types=[
            pltpu.VMEM((BPW * F,), jnp.int32),
            pltpu.VMEM((CB,), jnp.int32),
            pltpu.VMEM((CB,), jnp.int32),
            pltpu.VMEM((CB,), jnp.int32),
            pltpu.VMEM((CB,), jnp.int32),
            pltpu.VMEM((NSL, CB), jnp.int32),
            pltpu.VMEM((NSL, CB, 128), jnp.float32),
            pltpu.VMEM((NSL, D, CB), jnp.float32),
        ] + [pltpu.SemaphoreType.DMA] * 8,
        compiler_params=pltpu.CompilerParams(use_tc_tiling_on_sc=True,
                                             needs_layout_passes=False),
    )(xf, ptbl)


@jax.jit
def kernel(x, tables):
    xf = x.astype(jnp.int32).reshape(B * F)
    tbl_t = jnp.transpose(tables, (0, 2, 1)).reshape(F * D, V)
    ptbl = _pack(tbl_t)
    out_fdb = _gather(xf, ptbl)
    return jnp.transpose(out_fdb, (2, 0, 1))


# speedup vs baseline: 1.7624x; 1.1315x over previous
import jax
import jax.numpy as jnp
from jax import lax
from jax.experimental import pallas as pl
from jax.experimental.pallas import tpu as pltpu
from jax.experimental.pallas import tpu_sc as plsc

B = 16384
F = 26
V = 100000
D = 32

NC = 2
NS = 16
NW = NC * NS

V4 = V // 4
BPW = B // NW
CB = 128
NSL = 4


def _pack_body(t_ref, o_ref):
    t = t_ref[...]
    for q in range(4):
        o_ref[:, q * D:(q + 1) * D] = t[:, q * V4:(q + 1) * V4].T


def _pack(tbl_t, f0, nf):
    return pl.pallas_call(
        _pack_body,
        grid=(nf,),
        in_specs=[pl.BlockSpec((D, V), lambda f: (f0 + f, 0))],
        out_specs=pl.BlockSpec((V4, 128), lambda f: (f, 0)),
        out_shape=jax.ShapeDtypeStruct((nf * V4, 128), jnp.float32),
        compiler_params=pltpu.CompilerParams(
            vmem_limit_bytes=110 * 1024 * 1024),
    )(tbl_t)


def _gather_body(nf, f0, x_hbm, ptbl_hbm, out_hbm,
                 xk, idxb0, idxb1, idxb2, idxb3, laneb, gbuf, ebuf,
                 g0, g1, g2, g3, w0, w1, w2, w3):
    nt = nf * (BPW // CB)
    ngrp = nt // NSL
    c = lax.axis_index("c")
    s = lax.axis_index("s")
    wid = s * NC + c
    b0 = wid * BPW

    pltpu.sync_copy(x_hbm.at[pl.ds(b0 * F, BPW * F)], xk)

    iota = lax.iota(jnp.int32, 16)
    gsems = (g0, g1, g2, g3)
    wsems = (w0, w1, w2, w3)
    idxbs = (idxb0, idxb1, idxb2, idxb3)

    def build(t, slot):
        fl = t // NSL
        h = lax.rem(t, NSL)

        def grp(g, carry):
            j = h * CB + g * 16 + iota
            v = plsc.load_gather(xk, [f0 + fl + F * j])
            idxbs[slot][pl.ds(g * 16, 16)] = fl * V4 + lax.rem(v, V4)
            laneb[slot, pl.ds(g * 16, 16)] = lax.div(v, V4) * D
            return carry

        lax.fori_loop(0, CB // 16, grp, 0)

    def gstart(slot):
        pltpu.async_copy(ptbl_hbm.at[idxbs[slot]], gbuf.at[slot],
                         gsems[slot])

    def gwait(slot):
        pltpu.make_async_copy(ptbl_hbm.at[idxbs[slot]], gbuf.at[slot],
                              gsems[slot]).wait()

    def extract(slot):
        def grp(g, carry):
            j = g * 16 + iota
            lj = laneb[slot, pl.ds(g * 16, 16)]
            for d in range(D):
                ebuf[slot, d, pl.ds(g * 16, 16)] = \
                    plsc.load_gather(gbuf.at[slot], [j, lj + d])
            return carry

        lax.fori_loop(0, CB // 16, grp, 0)

    def wstart(t, slot):
        fl = t // NSL
        h = lax.rem(t, NSL)
        pltpu.async_copy(ebuf.at[slot],
                         out_hbm.at[fl, :, pl.ds(b0 + h * CB, CB)],
                         wsems[slot])

    def wwait(slot):
        pltpu.make_async_copy(ebuf.at[slot],
                              out_hbm.at[0, :, pl.ds(b0, CB)],
                              wsems[slot]).wait()

    for sl in range(NSL):
        build(sl, sl)
        gstart(sl)

    def group(gi, carry):
        for sl in range(NSL):
            t = NSL * gi + sl
            gwait(sl)

            @pl.when(gi >= 1)
            def _():
                wwait(sl)

            extract(sl)
            wstart(t, sl)

            @pl.when(gi < ngrp - 1)
            def _():
                build(t + NSL, sl)
                gstart(sl)

        return carry

    lax.fori_loop(0, ngrp, group, 0)
    for sl in range(NSL):
        wwait(sl)


def _gather(xf, ptbl, f0, nf):
    mesh = plsc.VectorSubcoreMesh(core_axis_name="c", subcore_axis_name="s",
                                  num_cores=NC, num_subcores=NS)

    def body(*refs):
        _gather_body(nf, f0, *refs)

    return pl.kernel(
        body,
        out_type=jax.ShapeDtypeStruct((nf, D, B), jnp.float32),
        mesh=mesh,
        scratch_types=[
            pltpu.VMEM((BPW * F,), jnp.int32),
            pltpu.VMEM((CB,), jnp.int32),
            pltpu.VMEM((CB,), jnp.int32),
            pltpu.VMEM((CB,), jnp.int32),
            pltpu.VMEM((CB,), jnp.int32),
            pltpu.VMEM((NSL, CB), jnp.int32),
            pltpu.VMEM((NSL, CB, 128), jnp.float32),
            pltpu.VMEM((NSL, D, CB), jnp.float32),
        ] + [pltpu.SemaphoreType.DMA] * 8,
        compiler_params=pltpu.CompilerParams(use_tc_tiling_on_sc=True,
                                             needs_layout_passes=False),
    )(xf, ptbl)


@jax.jit
def kernel(x, tables):
    xf = x.astype(jnp.int32).reshape(B * F)
    tbl_t = jnp.transpose(tables, (0, 2, 1)).reshape(F * D, V)
    halves = []
    for f0, nf in ((0, 13), (13, 13)):
        ptbl = _pack(tbl_t, f0, nf)
        halves.append(_gather(xf, ptbl, f0, nf))
    out_fdb = jnp.concatenate(halves, axis=0)
    return jnp.transpose(out_fdb, (2, 0, 1))


# speedup vs baseline: 1.9098x; 1.0837x over previous
import jax
import jax.numpy as jnp
from jax import lax
from jax.experimental import pallas as pl
from jax.experimental.pallas import tpu as pltpu
from jax.experimental.pallas import tpu_sc as plsc

B = 16384
F = 26
V = 100000
D = 32

NC = 2
NS = 16
NW = NC * NS

V4 = V // 4
BPW = B // NW
CB = 128
NSL = 4


def _pack_body(t_ref, o_ref):
    t = t_ref[...]
    for q in range(4):
        o_ref[:, q * D:(q + 1) * D] = t[:, q * V4:(q + 1) * V4].T


def _pack(tbl_t, f0, nf):
    return pl.pallas_call(
        _pack_body,
        grid=(nf,),
        in_specs=[pl.BlockSpec((D, V), lambda f: (f0 + f, 0))],
        out_specs=pl.BlockSpec((V4, 128), lambda f: (f, 0)),
        out_shape=jax.ShapeDtypeStruct((nf * V4, 128), jnp.float32),
        compiler_params=pltpu.CompilerParams(
            vmem_limit_bytes=110 * 1024 * 1024),
    )(tbl_t)


def _gather_body(nf, f0, x_hbm, ptbl_hbm, out_hbm,
                 xk, idxb0, idxb1, idxb2, idxb3, laneb, gbuf, ebuf,
                 g0, g1, g2, g3, w0, w1, w2, w3):
    nt = nf * (BPW // CB)
    ngrp = nt // NSL
    c = lax.axis_index("c")
    s = lax.axis_index("s")
    wid = s * NC + c
    b0 = wid * BPW

    pltpu.sync_copy(x_hbm.at[pl.ds(b0 * F, BPW * F)], xk)

    iota = lax.iota(jnp.int32, 16)
    gsems = (g0, g1, g2, g3)
    wsems = (w0, w1, w2, w3)
    idxbs = (idxb0, idxb1, idxb2, idxb3)

    def build(t, slot):
        fl = t // NSL
        h = lax.rem(t, NSL)

        def grp(g, carry):
            j = h * CB + g * 16 + iota
            v = plsc.load_gather(xk, [f0 + fl + F * j])
            idxbs[slot][pl.ds(g * 16, 16)] = fl * V4 + lax.rem(v, V4)
            laneb[slot, pl.ds(g * 16, 16)] = lax.div(v, V4) * D
            return carry

        lax.fori_loop(0, CB // 16, grp, 0)

    def gstart(slot):
        pltpu.async_copy(ptbl_hbm.at[idxbs[slot]], gbuf.at[slot],
                         gsems[slot])

    def gwait(slot):
        pltpu.make_async_copy(ptbl_hbm.at[idxbs[slot]], gbuf.at[slot],
                              gsems[slot]).wait()

    def extract(slot):
        def grp(g, carry):
            j = g * 16 + iota
            lj = laneb[slot, pl.ds(g * 16, 16)]
            for d in range(D):
                ebuf[slot, d, pl.ds(g * 16, 16)] = \
                    plsc.load_gather(gbuf.at[slot], [j, lj + d])
            return carry

        lax.fori_loop(0, CB // 16, grp, 0)

    def wstart(t, slot):
        fl = t // NSL
        h = lax.rem(t, NSL)
        pltpu.async_copy(ebuf.at[slot],
                         out_hbm.at[fl, :, pl.ds(b0 + h * CB, CB)],
                         wsems[slot])

    def wwait(slot):
        pltpu.make_async_copy(ebuf.at[slot],
                              out_hbm.at[0, :, pl.ds(b0, CB)],
                              wsems[slot]).wait()

    for sl in range(NSL):
        build(sl, sl)
        gstart(sl)

    def group(gi, carry):
        for sl in range(NSL):
            t = NSL * gi + sl
            gwait(sl)

            @pl.when(gi >= 1)
            def _():
                wwait(sl)

            extract(sl)
            wstart(t, sl)

            @pl.when(gi < ngrp - 1)
            def _():
                build(t + NSL, sl)
                gstart(sl)

        return carry

    lax.fori_loop(0, ngrp, group, 0)
    for sl in range(NSL):
        wwait(sl)


def _gather(xf, ptbl, f0, nf):
    mesh = plsc.VectorSubcoreMesh(core_axis_name="c", subcore_axis_name="s",
                                  num_cores=NC, num_subcores=NS)

    def body(*refs):
        _gather_body(nf, f0, *refs)

    return pl.kernel(
        body,
        out_type=jax.ShapeDtypeStruct((nf, D, B), jnp.float32),
        mesh=mesh,
        scratch_types=[
            pltpu.VMEM((BPW * F,), jnp.int32),
            pltpu.VMEM((CB,), jnp.int32),
            pltpu.VMEM((CB,), jnp.int32),
            pltpu.VMEM((CB,), jnp.int32),
            pltpu.VMEM((CB,), jnp.int32),
            pltpu.VMEM((NSL, CB), jnp.int32),
            pltpu.VMEM((NSL, CB, 128), jnp.float32),
            pltpu.VMEM((NSL, D, CB), jnp.float32),
        ] + [pltpu.SemaphoreType.DMA] * 8,
        compiler_params=pltpu.CompilerParams(use_tc_tiling_on_sc=True,
                                             needs_layout_passes=False),
    )(xf, ptbl)


@jax.jit
def kernel(x, tables):
    xf = x.astype(jnp.int32).reshape(B * F)
    tbl_t = jnp.transpose(tables, (0, 2, 1)).reshape(F * D, V)
    halves = []
    for f0, nf in ((0, 7), (7, 7), (14, 7), (21, 5)):
        ptbl = _pack(tbl_t, f0, nf)
        halves.append(_gather(xf, ptbl, f0, nf))
    out_fdb = jnp.concatenate(halves, axis=0)
    return jnp.transpose(out_fdb, (2, 0, 1))
